# baseline (speedup 1.0000x reference)
import functools

import jax
import jax.numpy as jnp
from jax import lax
from jax.experimental import pallas as pl
from jax.experimental.pallas import tpu as pltpu
from jax.experimental.pallas import tpu_sc as plsc

N_LAST = 8


def kernel(x, seq_len):
    B, T, F = x.shape
    Fh = F // 2
    mesh = plsc.VectorSubcoreMesh(core_axis_name="c", subcore_axis_name="s")
    nc = mesh.num_cores

    @functools.partial(
        pl.kernel,
        out_type=jax.ShapeDtypeStruct((B, N_LAST, F), x.dtype),
        mesh=mesh,
        scratch_types=[
            pltpu.VMEM((16,), jnp.int32),
            pltpu.VMEM((N_LAST, Fh), jnp.float32),
        ],
        compiler_params=pltpu.CompilerParams(
            use_tc_tiling_on_sc=False, needs_layout_passes=False
        ),
    )
    def take_last(x_hbm, seq_hbm, out_hbm, seq_v, buf_v):
        wid = lax.axis_index("s") * nc + lax.axis_index("c")
        b = wid // 2
        h = wid % 2
        pltpu.sync_copy(seq_hbm, seq_v)
        lens = seq_v[...]
        lane = lax.iota(jnp.int32, 16)
        start = jnp.max(jnp.where(lane == b, lens, 0)) - N_LAST
        pltpu.sync_copy(
            x_hbm.at[b, pl.ds(start, N_LAST), pl.ds(h * Fh, Fh)], buf_v
        )
        pltpu.sync_copy(buf_v, out_hbm.at[b, :, pl.ds(h * Fh, Fh)])

    return take_last(x, seq_len)


# speedup vs baseline: 5.5418x; 5.5418x over previous
import functools

import jax
import jax.numpy as jnp
from jax import lax
from jax.experimental import pallas as pl
from jax.experimental.pallas import tpu as pltpu
from jax.experimental.pallas import tpu_sc as plsc

N_LAST = 8


def kernel(x, seq_len):
    B, T, F = x.shape
    x2d = x.reshape(B * T, F)
    mesh = plsc.VectorSubcoreMesh(core_axis_name="c", subcore_axis_name="s")
    nc = mesh.num_cores

    @functools.partial(
        pl.kernel,
        out_type=jax.ShapeDtypeStruct((B * N_LAST, F), x.dtype),
        mesh=mesh,
        scratch_types=[
            pltpu.VMEM((16,), jnp.int32),
            pltpu.VMEM((N_LAST,), jnp.int32),
            pltpu.VMEM((N_LAST, F), jnp.float32),
            pltpu.SemaphoreType.DMA,
        ],
        compiler_params=pltpu.CompilerParams(needs_layout_passes=False),
    )
    def take_last(x_hbm, seq_hbm, out_hbm, seq_v, idx_v, rows_v, sem):
        wid = lax.axis_index("s") * nc + lax.axis_index("c")

        @pl.when(wid < B)
        def _():
            b = wid
            pltpu.sync_copy(seq_hbm, seq_v)
            lane = lax.iota(jnp.int32, 16)
            len_b = plsc.load_gather(seq_v, [jnp.full((16,), b, jnp.int32)])
            row_ids = b * T - N_LAST + len_b + lane
            plsc.store_scatter(idx_v, [lane], row_ids, mask=lane < N_LAST)
            pltpu.async_copy(x_hbm.at[idx_v], rows_v, sem).wait()
            pltpu.sync_copy(
                rows_v, out_hbm.at[pl.ds(pl.multiple_of(b * N_LAST, 8), N_LAST)]
            )

    out = take_last(x2d, seq_len)
    return out.reshape(B, N_LAST, F)


# speedup vs baseline: 5.9439x; 1.0726x over previous
import functools

import jax
import jax.numpy as jnp
from jax import lax
from jax.experimental import pallas as pl
from jax.experimental.pallas import tpu as pltpu
from jax.experimental.pallas import tpu_sc as plsc

N_LAST = 8


def kernel(x, seq_len):
    B, T, F = x.shape
    x2d = x.reshape(B * T, F)
    mesh = plsc.VectorSubcoreMesh(
        core_axis_name="c", subcore_axis_name="s", num_cores=1
    )

    @functools.partial(
        pl.kernel,
        out_type=jax.ShapeDtypeStruct((B * N_LAST, F), x.dtype),
        mesh=mesh,
        scratch_types=[
            pltpu.VMEM((16,), jnp.int32),
            pltpu.VMEM((N_LAST,), jnp.int32),
            pltpu.VMEM((N_LAST, F), jnp.float32),
            pltpu.SemaphoreType.DMA,
        ],
        compiler_params=pltpu.CompilerParams(needs_layout_passes=False),
    )
    def take_last(x_hbm, seq_hbm, out_hbm, seq_v, idx_v, rows_v, sem):
        b = lax.axis_index("s")
        pltpu.sync_copy(seq_hbm, seq_v)
        lane = lax.iota(jnp.int32, 16)
        len_b = plsc.load_gather(seq_v, [jnp.full((16,), b, jnp.int32)])
        row_ids = b * T - N_LAST + len_b + lane
        plsc.store_scatter(idx_v, [lane], row_ids, mask=lane < N_LAST)
        pltpu.async_copy(x_hbm.at[idx_v], rows_v, sem).wait()
        pltpu.sync_copy(
            rows_v, out_hbm.at[pl.ds(pl.multiple_of(b * N_LAST, 8), N_LAST)]
        )

    out = take_last(x2d, seq_len)
    return out.reshape(B, N_LAST, F)


# speedup vs baseline: 5.9655x; 1.0036x over previous
import functools

import jax
import jax.numpy as jnp
from jax import lax
from jax.experimental import pallas as pl
from jax.experimental.pallas import tpu as pltpu
from jax.experimental.pallas import tpu_sc as plsc

N_LAST = 8


def kernel(x, seq_len):
    B, T, F = x.shape
    x2d = x.reshape(B * T, F)
    mesh = plsc.VectorSubcoreMesh(
        core_axis_name="c", subcore_axis_name="s", num_cores=1
    )

    @functools.partial(
        pl.kernel,
        out_type=jax.ShapeDtypeStruct((B * N_LAST, F), x.dtype),
        mesh=mesh,
        scratch_types=[
            pltpu.VMEM((16,), jnp.int32),
            pltpu.VMEM((N_LAST,), jnp.int32),
            pltpu.VMEM((N_LAST, F), jnp.float32),
            pltpu.SemaphoreType.DMA,
        ],
        compiler_params=pltpu.CompilerParams(
            needs_layout_passes=False, skip_device_barrier=True
        ),
    )
    def take_last(x_hbm, seq_hbm, out_hbm, seq_v, idx_v, rows_v, sem):
        b = lax.axis_index("s")
        pltpu.sync_copy(seq_hbm, seq_v)
        lane = lax.iota(jnp.int32, 16)
        len_b = plsc.load_gather(seq_v, [jnp.full((16,), b, jnp.int32)])
        row_ids = b * T - N_LAST + len_b + lane
        plsc.store_scatter(idx_v, [lane], row_ids, mask=lane < N_LAST)
        pltpu.async_copy(x_hbm.at[idx_v], rows_v, sem).wait()
        pltpu.sync_copy(
            rows_v, out_hbm.at[pl.ds(pl.multiple_of(b * N_LAST, 8), N_LAST)]
        )

    out = take_last(x2d, seq_len)
    return out.reshape(B, N_LAST, F)
